# Initial kernel scaffold; baseline (speedup 1.0000x reference)
import jax
import jax.numpy as jnp
from jax.experimental import pallas as pl


def kernel(x, x_embed, shared_prompt, task_prompts_table):
    raise NotImplementedError("write your pallas kernel here")


# speedup vs baseline: 1.0287x; 1.0287x over previous
import functools

import jax
import jax.numpy as jnp
from jax import lax
from jax.experimental import pallas as pl
from jax.experimental.pallas import tpu as pltpu
from jax.experimental.pallas import tpu_sc as plsc

_LENGTH = 128
_NUM_TASKS = 64
_D_MODEL = 1024
_BATCH = 32
_TASK_SIZE = _LENGTH * _D_MODEL

_CH = 2048
_K = _TASK_SIZE // _CH
_G = 16
_NGROUPS = _K // _G

_NC = 2
_NS = 16


def _body(idx_hbm, table_hbm, shared_hbm, out_hbm,
          idx16_v, rowidx_v, rows_v, sh_v, sem):
    wid = lax.axis_index("s") * _NC + lax.axis_index("c")
    pltpu.async_copy(
        idx_hbm.at[jnp.full((16,), wid, jnp.int32)], idx16_v, sem
    ).wait()
    task = idx16_v[...]
    for g in range(_NGROUPS):
        rowidx_v[...] = task * _K + (g * _G + lax.iota(jnp.int32, 16))
        gather = pltpu.async_copy(table_hbm.at[rowidx_v], rows_v, sem)
        pltpu.sync_copy(shared_hbm.at[pl.ds(g * _G, _G)], sh_v)
        gather.wait()

        def add_cols(c, _):
            for r in range(_G):
                sl = pl.ds(c * 16, 16)
                rows_v.at[r][sl] = rows_v.at[r][sl] + sh_v.at[r][sl]
            return 0

        lax.fori_loop(0, _CH // 16, add_cols, 0)
        pltpu.sync_copy(rows_v, out_hbm.at[pl.ds(wid * _K + g * _G, _G)])


@jax.jit
def _sc_prompt(task_idx, table2, shared2):
    mesh = plsc.VectorSubcoreMesh(core_axis_name="c", subcore_axis_name="s")
    return pl.kernel(
        _body,
        out_type=jax.ShapeDtypeStruct((_BATCH * _K, _CH), jnp.float32),
        mesh=mesh,
        scratch_types=[
            pltpu.VMEM((16,), jnp.int32),
            pltpu.VMEM((16,), jnp.int32),
            pltpu.VMEM((_G, _CH), jnp.float32),
            pltpu.VMEM((_G, _CH), jnp.float32),
            pltpu.SemaphoreType.DMA,
        ],
    )(task_idx, table2, shared2)


def kernel(x, x_embed, shared_prompt, task_prompts_table):
    task_idx = x[:, 0].astype(jnp.int32)
    table2 = task_prompts_table.reshape(_NUM_TASKS * _K, _CH)
    shared2 = shared_prompt.reshape(_K, _CH)
    out = _sc_prompt(task_idx, table2, shared2)
    return out.reshape(_BATCH, _LENGTH, _D_MODEL)


# speedup vs baseline: 1.5441x; 1.5011x over previous
import functools

import jax
import jax.numpy as jnp
from jax import lax
from jax.experimental import pallas as pl
from jax.experimental.pallas import tpu as pltpu
from jax.experimental.pallas import tpu_sc as plsc

_LENGTH = 128
_NUM_TASKS = 64
_D_MODEL = 1024
_BATCH = 32
_TASK_SIZE = _LENGTH * _D_MODEL

_R = 16
_NGROUPS = _LENGTH // _R

_NC = 2
_NS = 16


def _body(idx_hbm, table_hbm, shared_hbm, out_hbm, idx16_v, rows_v, sh_v, sem):
    wid = lax.axis_index("s") * _NC + lax.axis_index("c")
    pltpu.async_copy(
        idx_hbm.at[jnp.full((16,), wid, jnp.int32)], idx16_v, sem
    ).wait()
    task = idx16_v[...][0]
    for g in range(_NGROUPS):
        copies = []
        for r in range(_R):
            copies.append(pltpu.async_copy(
                table_hbm.at[pl.ds(task, 1),
                             pl.ds((g * _R + r) * _D_MODEL, _D_MODEL)],
                rows_v.at[pl.ds(r, 1), :], sem))
        pltpu.sync_copy(shared_hbm.at[pl.ds(g * _R, _R), :], sh_v)
        for c in copies:
            c.wait()

        def add_cols(c, _):
            for r in range(_R):
                sl = pl.ds(c * 16, 16)
                sh_v.at[r][sl] = sh_v.at[r][sl] + rows_v.at[r][sl]
            return 0

        lax.fori_loop(0, _D_MODEL // 16, add_cols, 0)
        pltpu.sync_copy(sh_v, out_hbm.at[pl.ds(wid * _LENGTH + g * _R, _R), :])


@jax.jit
def _sc_prompt(task_idx, table, shared):
    mesh = plsc.VectorSubcoreMesh(core_axis_name="c", subcore_axis_name="s")
    return pl.kernel(
        _body,
        out_type=jax.ShapeDtypeStruct((_BATCH * _LENGTH, _D_MODEL), jnp.float32),
        mesh=mesh,
        scratch_types=[
            pltpu.VMEM((16,), jnp.int32),
            pltpu.VMEM((_R, _D_MODEL), jnp.float32),
            pltpu.VMEM((_R, _D_MODEL), jnp.float32),
            pltpu.SemaphoreType.DMA,
        ],
    )(task_idx, table, shared)


def kernel(x, x_embed, shared_prompt, task_prompts_table):
    task_idx = x[:, 0].astype(jnp.int32)
    out = _sc_prompt(task_idx, task_prompts_table, shared_prompt)
    return out.reshape(_BATCH, _LENGTH, _D_MODEL)
